# baseline (speedup 1.0000x reference)
import functools

import jax
import jax.numpy as jnp
from jax import lax
from jax.experimental import pallas as pl
from jax.experimental.pallas import tpu as pltpu
from jax.experimental.pallas import tpu_sc as plsc

T = 2048
D = 1024
F = 1024
E = 8
SOFTCAP = 30.0

BT = 256
NB = 24
S = NB * BT

NW = 32
DISPATCH_CHUNK = 64
ROWS_PER_W = S // NW
COMB_CHUNK = 16
TOK_PER_W = T // NW

_LANES = 128
_NEG = -1e30


def _router_body(x_ref, gw_ref, e1_ref, e2_ref, w1_ref, w2_ref):
    x = x_ref[...]
    gw = gw_ref[...]
    logits = jnp.dot(x, gw, preferred_element_type=jnp.float32)
    l = jnp.tanh(logits / SOFTCAP)
    lane = lax.broadcasted_iota(jnp.int32, l.shape, 1)
    l = jnp.where(lane < E, l, _NEG)
    m1 = jnp.max(l, axis=1, keepdims=True)
    i1 = jnp.min(jnp.where(l == m1, lane, _LANES), axis=1, keepdims=True)
    l2 = jnp.where(lane == i1, _NEG, l)
    m2 = jnp.max(l2, axis=1, keepdims=True)
    i2 = jnp.min(jnp.where(l2 == m2, lane, _LANES), axis=1, keepdims=True)
    w1 = 1.0 / (1.0 + jnp.exp(m2 - m1))
    w2 = 1.0 - w1
    shp = l.shape
    e1_ref[...] = jnp.broadcast_to(i1, shp)
    e2_ref[...] = jnp.broadcast_to(i2, shp)
    w1_ref[...] = jnp.broadcast_to(w1, shp)
    w2_ref[...] = jnp.broadcast_to(w2, shp)


def _router(x, gate_w):
    btr = 256
    gw_pad = jnp.zeros((D, _LANES), jnp.float32).at[:, :E].set(gate_w)
    outs = pl.pallas_call(
        _router_body,
        grid=(T // btr,),
        in_specs=[
            pl.BlockSpec((btr, D), lambda b: (b, 0)),
            pl.BlockSpec((D, _LANES), lambda b: (0, 0)),
        ],
        out_specs=[
            pl.BlockSpec((btr, _LANES), lambda b: (b, 0)),
            pl.BlockSpec((btr, _LANES), lambda b: (b, 0)),
            pl.BlockSpec((btr, _LANES), lambda b: (b, 0)),
            pl.BlockSpec((btr, _LANES), lambda b: (b, 0)),
        ],
        out_shape=[
            jax.ShapeDtypeStruct((T, _LANES), jnp.int32),
            jax.ShapeDtypeStruct((T, _LANES), jnp.int32),
            jax.ShapeDtypeStruct((T, _LANES), jnp.float32),
            jax.ShapeDtypeStruct((T, _LANES), jnp.float32),
        ],
    )(x, gw_pad)
    e1, e2, w1, w2 = (o[:, 0] for o in outs)
    return e1, e2, w1, w2


def _dispatch_indices(e1, e2, w1, w2):
    e_flat = jnp.stack([e1, e2], 1).reshape(-1).astype(jnp.int32)
    w_flat = jnp.stack([w1, w2], 1).reshape(-1)
    onehot = (e_flat[:, None] == jnp.arange(E, dtype=jnp.int32)[None, :]).astype(jnp.int32)
    cum = jnp.cumsum(onehot, axis=0)
    counts = cum[-1]
    rank = jnp.take_along_axis(cum, e_flat[:, None], axis=1)[:, 0] - 1
    nblk = -(-counts // BT)
    blk_cum = jnp.cumsum(nblk)
    pad_off = (blk_cum - nblk) * BT
    dst = (pad_off[e_flat] + rank).astype(jnp.int32)
    tok = (jnp.arange(2 * T, dtype=jnp.int32) // 2)
    src_tok = jnp.zeros(S, jnp.int32).at[dst].set(tok)
    w_d = jnp.zeros(S, jnp.float32).at[dst].set(w_flat)
    s0 = dst[0::2]
    s1 = dst[1::2]
    bidx = jnp.arange(NB, dtype=jnp.int32)
    block_expert = jnp.minimum(
        jnp.sum((bidx[:, None] >= blk_cum[None, :]).astype(jnp.int32), axis=1), E - 1
    ).astype(jnp.int32)
    w_bcast = jnp.broadcast_to(w_d[:, None], (S, _LANES))
    return src_tok, s0, s1, block_expert, w_bcast


def _sc_mesh():
    return plsc.VectorSubcoreMesh(core_axis_name="c", subcore_axis_name="s")


def _dispatch_body(x_hbm, srctok_hbm, xd_hbm, idx_v, rows_v, sem):
    wid = lax.axis_index("s") * 2 + lax.axis_index("c")
    base = wid * ROWS_PER_W
    for i in range(ROWS_PER_W // DISPATCH_CHUNK):
        off = base + i * DISPATCH_CHUNK
        pltpu.sync_copy(srctok_hbm.at[pl.ds(off, DISPATCH_CHUNK)], idx_v)
        pltpu.async_copy(x_hbm.at[idx_v], rows_v, sem).wait()
        pltpu.sync_copy(rows_v, xd_hbm.at[pl.ds(off, DISPATCH_CHUNK)])


def _dispatch_gather(x, src_tok):
    k = functools.partial(
        pl.kernel,
        out_type=jax.ShapeDtypeStruct((S, D), jnp.float32),
        mesh=_sc_mesh(),
        scratch_types=[
            pltpu.VMEM((DISPATCH_CHUNK,), jnp.int32),
            pltpu.VMEM((DISPATCH_CHUNK, D), jnp.float32),
            pltpu.SemaphoreType.DMA,
        ],
    )(_dispatch_body)
    return k(x, src_tok)


def _gmm_body(be_ref, x_ref, win_ref, wout_ref, ws_ref, y_ref):
    x = x_ref[...]
    h = jnp.dot(x, win_ref[0], preferred_element_type=jnp.float32)
    h = jax.nn.gelu(h)
    y = jnp.dot(h, wout_ref[0], preferred_element_type=jnp.float32)
    y_ref[...] = y * ws_ref[...][:, 0:1]


def _gmm(x_d, w_in, w_out, w_bcast, block_expert):
    grid_spec = pltpu.PrefetchScalarGridSpec(
        num_scalar_prefetch=1,
        grid=(NB,),
        in_specs=[
            pl.BlockSpec((BT, D), lambda b, be: (b, 0)),
            pl.BlockSpec((1, D, F), lambda b, be: (be[b], 0, 0)),
            pl.BlockSpec((1, F, D), lambda b, be: (be[b], 0, 0)),
            pl.BlockSpec((BT, _LANES), lambda b, be: (b, 0)),
        ],
        out_specs=pl.BlockSpec((BT, D), lambda b, be: (b, 0)),
    )
    return pl.pallas_call(
        _gmm_body,
        grid_spec=grid_spec,
        out_shape=jax.ShapeDtypeStruct((S, D), jnp.float32),
        compiler_params=pltpu.CompilerParams(
            dimension_semantics=("arbitrary",),
        ),
    )(block_expert, x_d, w_in, w_out, w_bcast)


def _combine_body(y_hbm, s0_hbm, s1_hbm, out_hbm, i0_v, i1_v, r0_v, r1_v, sem0, sem1):
    wid = lax.axis_index("s") * 2 + lax.axis_index("c")
    base = wid * TOK_PER_W
    for c in range(TOK_PER_W // COMB_CHUNK):
        off = base + c * COMB_CHUNK
        pltpu.sync_copy(s0_hbm.at[pl.ds(off, COMB_CHUNK)], i0_v)
        pltpu.sync_copy(s1_hbm.at[pl.ds(off, COMB_CHUNK)], i1_v)
        cp0 = pltpu.async_copy(y_hbm.at[i0_v], r0_v, sem0)
        cp1 = pltpu.async_copy(y_hbm.at[i1_v], r1_v, sem1)
        cp0.wait()
        cp1.wait()

        def add_body(j, _):
            for i in range(COMB_CHUNK):
                sl = pl.ds(j * 16, 16)
                r0_v[i, sl] = r0_v[i, sl] + r1_v[i, sl]
            return 0

        lax.fori_loop(0, D // 16, add_body, 0)
        pltpu.sync_copy(r0_v, out_hbm.at[pl.ds(off, COMB_CHUNK)])


def _combine(y_d, s0, s1):
    k = functools.partial(
        pl.kernel,
        out_type=jax.ShapeDtypeStruct((T, D), jnp.float32),
        mesh=_sc_mesh(),
        scratch_types=[
            pltpu.VMEM((COMB_CHUNK,), jnp.int32),
            pltpu.VMEM((COMB_CHUNK,), jnp.int32),
            pltpu.VMEM((COMB_CHUNK, D), jnp.float32),
            pltpu.VMEM((COMB_CHUNK, D), jnp.float32),
            pltpu.SemaphoreType.DMA,
            pltpu.SemaphoreType.DMA,
        ],
    )(_combine_body)
    return k(y_d, s0, s1)


def kernel(hidden_states, gate_w, w_in, w_out):
    x = hidden_states.astype(jnp.float32)
    e1, e2, w1, w2 = _router(x, gate_w)
    src_tok, s0, s1, block_expert, w_bcast = _dispatch_indices(e1, e2, w1, w2)
    x_d = _dispatch_gather(x, src_tok)
    y_d = _gmm(x_d, w_in, w_out, w_bcast, block_expert)
    return _combine(y_d, s0, s1)


# speedup vs baseline: 1.0209x; 1.0209x over previous
import functools

import jax
import jax.numpy as jnp
from jax import lax
from jax.experimental import pallas as pl
from jax.experimental.pallas import tpu as pltpu
from jax.experimental.pallas import tpu_sc as plsc

T = 2048
D = 1024
F = 1024
E = 8
SOFTCAP = 30.0

BT = 256
NB = 24
S = NB * BT

NW = 32
DISPATCH_CHUNK = 32
DISPATCH_NBUF = 3
ROWS_PER_W = S // NW
DISPATCH_NCH = ROWS_PER_W // DISPATCH_CHUNK
COMB_CHUNK = 16
TOK_PER_W = T // NW
COMB_NCH = TOK_PER_W // COMB_CHUNK

_LANES = 128
_NEG = -1e30


def _router_body(x_ref, gw_ref, e1_ref, e2_ref, w1_ref, w2_ref):
    x = x_ref[...]
    gw = gw_ref[...]
    logits = jnp.dot(x, gw, preferred_element_type=jnp.float32)
    l = jnp.tanh(logits / SOFTCAP)
    lane = lax.broadcasted_iota(jnp.int32, l.shape, 1)
    l = jnp.where(lane < E, l, _NEG)
    m1 = jnp.max(l, axis=1, keepdims=True)
    i1 = jnp.min(jnp.where(l == m1, lane, _LANES), axis=1, keepdims=True)
    l2 = jnp.where(lane == i1, _NEG, l)
    m2 = jnp.max(l2, axis=1, keepdims=True)
    i2 = jnp.min(jnp.where(l2 == m2, lane, _LANES), axis=1, keepdims=True)
    w1 = 1.0 / (1.0 + jnp.exp(m2 - m1))
    w2 = 1.0 - w1
    shp = l.shape
    e1_ref[...] = jnp.broadcast_to(i1, shp)
    e2_ref[...] = jnp.broadcast_to(i2, shp)
    w1_ref[...] = jnp.broadcast_to(w1, shp)
    w2_ref[...] = jnp.broadcast_to(w2, shp)


def _router(x, gate_w):
    btr = 256
    gw_pad = jnp.zeros((D, _LANES), jnp.float32).at[:, :E].set(gate_w)
    outs = pl.pallas_call(
        _router_body,
        grid=(T // btr,),
        in_specs=[
            pl.BlockSpec((btr, D), lambda b: (b, 0)),
            pl.BlockSpec((D, _LANES), lambda b: (0, 0)),
        ],
        out_specs=[
            pl.BlockSpec((btr, _LANES), lambda b: (b, 0)),
            pl.BlockSpec((btr, _LANES), lambda b: (b, 0)),
            pl.BlockSpec((btr, _LANES), lambda b: (b, 0)),
            pl.BlockSpec((btr, _LANES), lambda b: (b, 0)),
        ],
        out_shape=[
            jax.ShapeDtypeStruct((T, _LANES), jnp.int32),
            jax.ShapeDtypeStruct((T, _LANES), jnp.int32),
            jax.ShapeDtypeStruct((T, _LANES), jnp.float32),
            jax.ShapeDtypeStruct((T, _LANES), jnp.float32),
        ],
    )(x, gw_pad)
    e1, e2, w1, w2 = (o[:, 0] for o in outs)
    return e1, e2, w1, w2


def _dispatch_indices(e1, e2, w1, w2):
    e_flat = jnp.stack([e1, e2], 1).reshape(-1).astype(jnp.int32)
    w_flat = jnp.stack([w1, w2], 1).reshape(-1)
    onehot = (e_flat[:, None] == jnp.arange(E, dtype=jnp.int32)[None, :]).astype(jnp.int32)
    cum = jnp.cumsum(onehot, axis=0)
    counts = cum[-1]
    rank = jnp.take_along_axis(cum, e_flat[:, None], axis=1)[:, 0] - 1
    nblk = -(-counts // BT)
    blk_cum = jnp.cumsum(nblk)
    pad_off = (blk_cum - nblk) * BT
    dst = (pad_off[e_flat] + rank).astype(jnp.int32)
    tok = (jnp.arange(2 * T, dtype=jnp.int32) // 2)
    src_tok = jnp.zeros(S, jnp.int32).at[dst].set(tok)
    w_d = jnp.zeros(S, jnp.float32).at[dst].set(w_flat)
    s0 = dst[0::2]
    s1 = dst[1::2]
    bidx = jnp.arange(NB, dtype=jnp.int32)
    block_expert = jnp.minimum(
        jnp.sum((bidx[:, None] >= blk_cum[None, :]).astype(jnp.int32), axis=1), E - 1
    ).astype(jnp.int32)
    w_bcast = jnp.broadcast_to(w_d[:, None], (S, _LANES))
    return src_tok, s0, s1, block_expert, w_bcast


def _sc_mesh():
    return plsc.VectorSubcoreMesh(core_axis_name="c", subcore_axis_name="s")


def _dispatch_body(x_hbm, srctok_hbm, xd_hbm, idx_v, r0, r1, r2,
                   g0, g1, g2, w0, w1, w2):
    rows = (r0, r1, r2)
    gsem = (g0, g1, g2)
    wsem = (w0, w1, w2)
    wid = lax.axis_index("s") * 2 + lax.axis_index("c")
    base = wid * ROWS_PER_W
    pltpu.sync_copy(srctok_hbm.at[wid], idx_v)
    gcp, wcp = {}, {}

    def start_gather(c):
        b = c % DISPATCH_NBUF
        gcp[c] = pltpu.async_copy(x_hbm.at[idx_v.at[c]], rows[b], gsem[b])

    for c in range(DISPATCH_NBUF):
        start_gather(c)
    for c in range(DISPATCH_NCH):
        b = c % DISPATCH_NBUF
        gcp[c].wait()
        wcp[c] = pltpu.async_copy(
            rows[b], xd_hbm.at[pl.ds(base + c * DISPATCH_CHUNK, DISPATCH_CHUNK)], wsem[b])
        nc = c + DISPATCH_NBUF
        if nc < DISPATCH_NCH:
            wcp[c].wait()
            start_gather(nc)
    for c in range(max(0, DISPATCH_NCH - DISPATCH_NBUF), DISPATCH_NCH):
        wcp[c].wait()


def _dispatch_gather(x, src_tok):
    k = functools.partial(
        pl.kernel,
        out_type=jax.ShapeDtypeStruct((S, D), jnp.float32),
        mesh=_sc_mesh(),
        scratch_types=[
            pltpu.VMEM((DISPATCH_NCH, DISPATCH_CHUNK), jnp.int32),
            pltpu.VMEM((DISPATCH_CHUNK, D), jnp.float32),
            pltpu.VMEM((DISPATCH_CHUNK, D), jnp.float32),
            pltpu.VMEM((DISPATCH_CHUNK, D), jnp.float32),
            pltpu.SemaphoreType.DMA,
            pltpu.SemaphoreType.DMA,
            pltpu.SemaphoreType.DMA,
            pltpu.SemaphoreType.DMA,
            pltpu.SemaphoreType.DMA,
            pltpu.SemaphoreType.DMA,
        ],
    )(_dispatch_body)
    return k(x, src_tok.reshape(NW, DISPATCH_NCH, DISPATCH_CHUNK))


def _gmm_body(be_ref, x_ref, win_ref, wout_ref, ws_ref, y_ref):
    x = x_ref[...]
    h = jnp.dot(x, win_ref[0], preferred_element_type=jnp.float32)
    h = jax.nn.gelu(h)
    y = jnp.dot(h, wout_ref[0], preferred_element_type=jnp.float32)
    y_ref[...] = y * ws_ref[...][:, 0:1]


def _gmm(x_d, w_in, w_out, w_bcast, block_expert):
    grid_spec = pltpu.PrefetchScalarGridSpec(
        num_scalar_prefetch=1,
        grid=(NB,),
        in_specs=[
            pl.BlockSpec((BT, D), lambda b, be: (b, 0)),
            pl.BlockSpec((1, D, F), lambda b, be: (be[b], 0, 0)),
            pl.BlockSpec((1, F, D), lambda b, be: (be[b], 0, 0)),
            pl.BlockSpec((BT, _LANES), lambda b, be: (b, 0)),
        ],
        out_specs=pl.BlockSpec((BT, D), lambda b, be: (b, 0)),
    )
    return pl.pallas_call(
        _gmm_body,
        grid_spec=grid_spec,
        out_shape=jax.ShapeDtypeStruct((S, D), jnp.float32),
        compiler_params=pltpu.CompilerParams(
            dimension_semantics=("arbitrary",),
        ),
    )(block_expert, x_d, w_in, w_out, w_bcast)


def _combine_body(y_hbm, s0_hbm, s1_hbm, out_hbm, i0_v, i1_v,
                  r0a, r0b, r1a, r1b, g0a, g0b, g1a, g1b, wa, wb):
    r0 = (r0a, r0b)
    r1 = (r1a, r1b)
    g0sem = (g0a, g0b)
    g1sem = (g1a, g1b)
    wsem = (wa, wb)
    wid = lax.axis_index("s") * 2 + lax.axis_index("c")
    base = wid * TOK_PER_W
    pltpu.sync_copy(s0_hbm.at[wid], i0_v)
    pltpu.sync_copy(s1_hbm.at[wid], i1_v)
    g0cp, g1cp, wcp = {}, {}, {}

    def start_gathers(c):
        b = c & 1
        g0cp[c] = pltpu.async_copy(y_hbm.at[i0_v.at[c]], r0[b], g0sem[b])
        g1cp[c] = pltpu.async_copy(y_hbm.at[i1_v.at[c]], r1[b], g1sem[b])

    start_gathers(0)
    for c in range(COMB_NCH):
        b = c & 1
        g0cp[c].wait()
        g1cp[c].wait()
        if c + 1 < COMB_NCH:
            if c - 1 >= 0:
                wcp[c - 1].wait()
            start_gathers(c + 1)

        def add_body(j, _):
            for i in range(COMB_CHUNK):
                sl = pl.ds(j * 16, 16)
                r0[b][i, sl] = r0[b][i, sl] + r1[b][i, sl]
            return 0

        lax.fori_loop(0, D // 16, add_body, 0)
        wcp[c] = pltpu.async_copy(
            r0[b], out_hbm.at[pl.ds(base + c * COMB_CHUNK, COMB_CHUNK)], wsem[b])
    for c in range(max(0, COMB_NCH - 2), COMB_NCH):
        wcp[c].wait()


def _combine(y_d, s0, s1):
    k = functools.partial(
        pl.kernel,
        out_type=jax.ShapeDtypeStruct((T, D), jnp.float32),
        mesh=_sc_mesh(),
        scratch_types=[
            pltpu.VMEM((COMB_NCH, COMB_CHUNK), jnp.int32),
            pltpu.VMEM((COMB_NCH, COMB_CHUNK), jnp.int32),
            pltpu.VMEM((COMB_CHUNK, D), jnp.float32),
            pltpu.VMEM((COMB_CHUNK, D), jnp.float32),
            pltpu.VMEM((COMB_CHUNK, D), jnp.float32),
            pltpu.VMEM((COMB_CHUNK, D), jnp.float32),
            pltpu.SemaphoreType.DMA,
            pltpu.SemaphoreType.DMA,
            pltpu.SemaphoreType.DMA,
            pltpu.SemaphoreType.DMA,
            pltpu.SemaphoreType.DMA,
            pltpu.SemaphoreType.DMA,
        ],
    )(_combine_body)
    return k(y_d, s0.reshape(NW, COMB_NCH, COMB_CHUNK), s1.reshape(NW, COMB_NCH, COMB_CHUNK))


def kernel(hidden_states, gate_w, w_in, w_out):
    x = hidden_states.astype(jnp.float32)
    e1, e2, w1, w2 = _router(x, gate_w)
    src_tok, s0, s1, block_expert, w_bcast = _dispatch_indices(e1, e2, w1, w2)
    x_d = _dispatch_gather(x, src_tok)
    y_d = _gmm(x_d, w_in, w_out, w_bcast, block_expert)
    return _combine(y_d, s0, s1)


# speedup vs baseline: 1.6958x; 1.6610x over previous
import functools

import jax
import jax.numpy as jnp
from jax import lax
from jax.experimental import pallas as pl
from jax.experimental.pallas import tpu as pltpu
from jax.experimental.pallas import tpu_sc as plsc

T = 2048
D = 1024
F = 1024
E = 8
SOFTCAP = 30.0

BT = 256
NB = 24
S = NB * BT

NW = 32
DISPATCH_CHUNK = 32
DISPATCH_NBUF = 3
ROWS_PER_W = S // NW
DISPATCH_NCH = ROWS_PER_W // DISPATCH_CHUNK
COMB_CHUNK = 16
TOK_PER_W = T // NW
COMB_NCH = TOK_PER_W // COMB_CHUNK

_LANES = 128
_NEG = -1e30


def _router_body(x_ref, gw_ref, e1_ref, e2_ref, w1_ref, w2_ref):
    x = x_ref[...]
    gw = gw_ref[...]
    logits = jnp.dot(x, gw, preferred_element_type=jnp.float32)
    l = jnp.tanh(logits / SOFTCAP)
    lane = lax.broadcasted_iota(jnp.int32, l.shape, 1)
    l = jnp.where(lane < E, l, _NEG)
    m1 = jnp.max(l, axis=1, keepdims=True)
    i1 = jnp.min(jnp.where(l == m1, lane, _LANES), axis=1, keepdims=True)
    l2 = jnp.where(lane == i1, _NEG, l)
    m2 = jnp.max(l2, axis=1, keepdims=True)
    i2 = jnp.min(jnp.where(l2 == m2, lane, _LANES), axis=1, keepdims=True)
    w1 = 1.0 / (1.0 + jnp.exp(m2 - m1))
    w2 = 1.0 - w1
    shp = l.shape
    e1_ref[...] = jnp.broadcast_to(i1, shp)
    e2_ref[...] = jnp.broadcast_to(i2, shp)
    w1_ref[...] = jnp.broadcast_to(w1, shp)
    w2_ref[...] = jnp.broadcast_to(w2, shp)


def _router(x, gate_w):
    btr = 256
    gw_pad = jnp.zeros((D, _LANES), jnp.float32).at[:, :E].set(gate_w)
    outs = pl.pallas_call(
        _router_body,
        grid=(T // btr,),
        in_specs=[
            pl.BlockSpec((btr, D), lambda b: (b, 0)),
            pl.BlockSpec((D, _LANES), lambda b: (0, 0)),
        ],
        out_specs=[
            pl.BlockSpec((btr, _LANES), lambda b: (b, 0)),
            pl.BlockSpec((btr, _LANES), lambda b: (b, 0)),
            pl.BlockSpec((btr, _LANES), lambda b: (b, 0)),
            pl.BlockSpec((btr, _LANES), lambda b: (b, 0)),
        ],
        out_shape=[
            jax.ShapeDtypeStruct((T, _LANES), jnp.int32),
            jax.ShapeDtypeStruct((T, _LANES), jnp.int32),
            jax.ShapeDtypeStruct((T, _LANES), jnp.float32),
            jax.ShapeDtypeStruct((T, _LANES), jnp.float32),
        ],
    )(x, gw_pad)
    e1, e2, w1, w2 = (o[:, 0] for o in outs)
    return e1, e2, w1, w2


def _dispatch_indices(e1, e2, w1, w2):
    e_flat = jnp.stack([e1, e2], 1).reshape(-1).astype(jnp.int32)
    w_flat = jnp.stack([w1, w2], 1).reshape(-1)
    onehot = (e_flat[:, None] == jnp.arange(E, dtype=jnp.int32)[None, :]).astype(jnp.int32)
    cum = jnp.cumsum(onehot, axis=0)
    counts = cum[-1]
    rank = jnp.take_along_axis(cum, e_flat[:, None], axis=1)[:, 0] - 1
    nblk = -(-counts // BT)
    blk_cum = jnp.cumsum(nblk)
    pad_off = (blk_cum - nblk) * BT
    dst = (pad_off[e_flat] + rank).astype(jnp.int32)
    tok = (jnp.arange(2 * T, dtype=jnp.int32) // 2)
    src_tok = (jnp.arange(S, dtype=jnp.int32) % T).at[dst].set(tok)
    w_d = jnp.zeros(S, jnp.float32).at[dst].set(w_flat)
    s0 = dst[0::2]
    s1 = dst[1::2]
    bidx = jnp.arange(NB, dtype=jnp.int32)
    block_expert = jnp.minimum(
        jnp.sum((bidx[:, None] >= blk_cum[None, :]).astype(jnp.int32), axis=1), E - 1
    ).astype(jnp.int32)
    w_bcast = jnp.broadcast_to(w_d[:, None], (S, _LANES))
    return src_tok, s0, s1, block_expert, w_bcast


def _sc_mesh():
    return plsc.VectorSubcoreMesh(core_axis_name="c", subcore_axis_name="s")


def _dispatch_body(x_hbm, srctok_hbm, xd_hbm, idx_v, r0, r1, r2,
                   g0, g1, g2, w0, w1, w2):
    rows = (r0, r1, r2)
    gsem = (g0, g1, g2)
    wsem = (w0, w1, w2)
    wid = lax.axis_index("s") * 2 + lax.axis_index("c")
    base = wid * ROWS_PER_W
    pltpu.sync_copy(srctok_hbm.at[wid], idx_v)
    gcp, wcp = {}, {}

    def start_gather(c):
        b = c % DISPATCH_NBUF
        gcp[c] = pltpu.async_copy(x_hbm.at[idx_v.at[c]], rows[b], gsem[b])

    for c in range(DISPATCH_NBUF):
        start_gather(c)
    for c in range(DISPATCH_NCH):
        b = c % DISPATCH_NBUF
        gcp[c].wait()
        wcp[c] = pltpu.async_copy(
            rows[b], xd_hbm.at[pl.ds(base + c * DISPATCH_CHUNK, DISPATCH_CHUNK)], wsem[b])
        nc = c + DISPATCH_NBUF
        if nc < DISPATCH_NCH:
            wcp[c].wait()
            start_gather(nc)
    for c in range(max(0, DISPATCH_NCH - DISPATCH_NBUF), DISPATCH_NCH):
        wcp[c].wait()


def _dispatch_gather(x, src_tok):
    k = functools.partial(
        pl.kernel,
        out_type=jax.ShapeDtypeStruct((S, D), jnp.float32),
        mesh=_sc_mesh(),
        scratch_types=[
            pltpu.VMEM((DISPATCH_NCH, DISPATCH_CHUNK), jnp.int32),
            pltpu.VMEM((DISPATCH_CHUNK, D), jnp.float32),
            pltpu.VMEM((DISPATCH_CHUNK, D), jnp.float32),
            pltpu.VMEM((DISPATCH_CHUNK, D), jnp.float32),
            pltpu.SemaphoreType.DMA,
            pltpu.SemaphoreType.DMA,
            pltpu.SemaphoreType.DMA,
            pltpu.SemaphoreType.DMA,
            pltpu.SemaphoreType.DMA,
            pltpu.SemaphoreType.DMA,
        ],
    )(_dispatch_body)
    return k(x, src_tok.reshape(NW, DISPATCH_NCH, DISPATCH_CHUNK))


def _gmm_body(be_ref, x_ref, win_ref, wout_ref, ws_ref, y_ref):
    x = x_ref[...]
    h = jnp.dot(x, win_ref[0], preferred_element_type=jnp.float32)
    h = jax.nn.gelu(h)
    y = jnp.dot(h, wout_ref[0], preferred_element_type=jnp.float32)
    y_ref[...] = y * ws_ref[...][:, 0:1]


def _gmm(x_d, w_in, w_out, w_bcast, block_expert):
    grid_spec = pltpu.PrefetchScalarGridSpec(
        num_scalar_prefetch=1,
        grid=(NB,),
        in_specs=[
            pl.BlockSpec((BT, D), lambda b, be: (b, 0)),
            pl.BlockSpec((1, D, F), lambda b, be: (be[b], 0, 0)),
            pl.BlockSpec((1, F, D), lambda b, be: (be[b], 0, 0)),
            pl.BlockSpec((BT, _LANES), lambda b, be: (b, 0)),
        ],
        out_specs=pl.BlockSpec((BT, D), lambda b, be: (b, 0)),
    )
    return pl.pallas_call(
        _gmm_body,
        grid_spec=grid_spec,
        out_shape=jax.ShapeDtypeStruct((S, D), jnp.float32),
        compiler_params=pltpu.CompilerParams(
            dimension_semantics=("arbitrary",),
        ),
    )(block_expert, x_d, w_in, w_out, w_bcast)


def _combine_body(y_hbm, s0_hbm, s1_hbm, out_hbm, i0_v, i1_v,
                  r0a, r0b, r1a, r1b, g0a, g0b, g1a, g1b, wa, wb):
    r0 = (r0a, r0b)
    r1 = (r1a, r1b)
    g0sem = (g0a, g0b)
    g1sem = (g1a, g1b)
    wsem = (wa, wb)
    wid = lax.axis_index("s") * 2 + lax.axis_index("c")
    base = wid * TOK_PER_W
    pltpu.sync_copy(s0_hbm.at[wid], i0_v)
    pltpu.sync_copy(s1_hbm.at[wid], i1_v)
    g0cp, g1cp, wcp = {}, {}, {}

    def start_gathers(c):
        b = c & 1
        g0cp[c] = pltpu.async_copy(y_hbm.at[i0_v.at[c]], r0[b], g0sem[b])
        g1cp[c] = pltpu.async_copy(y_hbm.at[i1_v.at[c]], r1[b], g1sem[b])

    start_gathers(0)
    for c in range(COMB_NCH):
        b = c & 1
        g0cp[c].wait()
        g1cp[c].wait()
        if c + 1 < COMB_NCH:
            if c - 1 >= 0:
                wcp[c - 1].wait()
            start_gathers(c + 1)

        def add_body(j, _):
            for i in range(COMB_CHUNK):
                sl = pl.ds(j * 16, 16)
                r0[b][i, sl] = r0[b][i, sl] + r1[b][i, sl]
            return 0

        lax.fori_loop(0, D // 16, add_body, 0)
        wcp[c] = pltpu.async_copy(
            r0[b], out_hbm.at[pl.ds(base + c * COMB_CHUNK, COMB_CHUNK)], wsem[b])
    for c in range(max(0, COMB_NCH - 2), COMB_NCH):
        wcp[c].wait()


def _combine(y_d, s0, s1):
    k = functools.partial(
        pl.kernel,
        out_type=jax.ShapeDtypeStruct((T, D), jnp.float32),
        mesh=_sc_mesh(),
        scratch_types=[
            pltpu.VMEM((COMB_NCH, COMB_CHUNK), jnp.int32),
            pltpu.VMEM((COMB_NCH, COMB_CHUNK), jnp.int32),
            pltpu.VMEM((COMB_CHUNK, D), jnp.float32),
            pltpu.VMEM((COMB_CHUNK, D), jnp.float32),
            pltpu.VMEM((COMB_CHUNK, D), jnp.float32),
            pltpu.VMEM((COMB_CHUNK, D), jnp.float32),
            pltpu.SemaphoreType.DMA,
            pltpu.SemaphoreType.DMA,
            pltpu.SemaphoreType.DMA,
            pltpu.SemaphoreType.DMA,
            pltpu.SemaphoreType.DMA,
            pltpu.SemaphoreType.DMA,
        ],
    )(_combine_body)
    return k(y_d, s0.reshape(NW, COMB_NCH, COMB_CHUNK), s1.reshape(NW, COMB_NCH, COMB_CHUNK))


def kernel(hidden_states, gate_w, w_in, w_out):
    x = hidden_states.astype(jnp.float32)
    e1, e2, w1, w2 = _router(x, gate_w)
    src_tok, s0, s1, block_expert, w_bcast = _dispatch_indices(e1, e2, w1, w2)
    x_d = _dispatch_gather(x, src_tok)
    y_d = _gmm(x_d, w_in, w_out, w_bcast, block_expert)
    return _combine(y_d, s0, s1)
